# Initial kernel scaffold; baseline (speedup 1.0000x reference)
import jax
import jax.numpy as jnp
from jax.experimental import pallas as pl


def kernel(node_features, edge_index, edge_sh, edge_basis, W1, b1, W2, b2, W3, b3, W_si, bn_weight, bn_bias):
    raise NotImplementedError("write your pallas kernel here")


# speedup vs baseline: 3.4027x; 3.4027x over previous
import functools

import jax
import jax.numpy as jnp
from jax import lax
from jax.experimental import pallas as pl
from jax.experimental.pallas import tpu as pltpu
from jax.experimental.pallas import tpu_sc as plsc

_MUL = 16
_NB = 8
_HID = 64
_WN = 256
_N_NODES = 10000
_E = 160000

_NW = 32
_CHUNK = 128
_NCH = 40
_EPW = _CHUNK * _NCH
_EPAD = _NW * _EPW
_BLK = 2048
_RPS = 640
_NPAD = 16 * _RPS

@functools.cache
def _build_sc_gather():
    mesh = plsc.VectorSubcoreMesh(core_axis_name="c", subcore_axis_name="s")
    return functools.partial(
        pl.kernel,
        out_type=jax.ShapeDtypeStruct((_NW, _EPW, _MUL), jnp.float32),
        mesh=mesh,
        compiler_params=pltpu.CompilerParams(use_tc_tiling_on_sc=False),
        scratch_types=[
            pltpu.VMEM((_EPW,), jnp.int32),
            pltpu.VMEM((_EPW, _MUL), jnp.float32),
            pltpu.SemaphoreType.DMA,
        ],
    )(_sc_gather_body)


def _sc_gather_body(node_hbm, src_hbm, x_hbm, idx_v, xbuf, sem):
    c = lax.axis_index("c")
    s = lax.axis_index("s")
    wid = s * 2 + c
    pltpu.sync_copy(src_hbm.at[wid], idx_v)
    pltpu.async_copy(node_hbm.at[idx_v], xbuf, sem).wait()
    pltpu.sync_copy(xbuf, x_hbm.at[wid])


@functools.cache
def _build_sc_scatter():
    mesh = plsc.VectorSubcoreMesh(core_axis_name="c", subcore_axis_name="s")
    return functools.partial(
        pl.kernel,
        out_type=jax.ShapeDtypeStruct((2, _NPAD, _MUL), jnp.float32),
        mesh=mesh,
        compiler_params=pltpu.CompilerParams(use_tc_tiling_on_sc=False),
        scratch_types=[
            pltpu.VMEM((_NCH, _CHUNK), jnp.int32),
            pltpu.VMEM((_EPW, _MUL), jnp.float32),
            pltpu.VMEM_SHARED((_NPAD, _MUL), jnp.float32),
            pltpu.SemaphoreType.DMA,
        ],
    )(_sc_scatter_body)


def _sc_scatter_body(msg_hbm, dst_hbm, zeros_hbm, out_hbm, idx_v, mbuf, agg_sh, sem):
    c = lax.axis_index("c")
    s = lax.axis_index("s")
    wid = s * 2 + c
    pltpu.sync_copy(zeros_hbm.at[pl.ds(s * _RPS, _RPS)],
                    agg_sh.at[pl.ds(s * _RPS, _RPS)])
    pltpu.sync_copy(dst_hbm.at[wid], idx_v)
    pltpu.sync_copy(msg_hbm.at[wid], mbuf)
    plsc.subcore_barrier()

    def fire(j, carry):
        pltpu.async_copy(
            mbuf.at[pl.ds(j * _CHUNK, _CHUNK)],
            agg_sh.at[idx_v.at[j]],
            sem,
            add=True,
        )
        return carry

    lax.fori_loop(0, _NCH, fire, 0)

    def drain(j, carry):
        pltpu.make_async_copy(
            mbuf.at[pl.ds(j * _CHUNK, _CHUNK)],
            agg_sh.at[idx_v.at[j]],
            sem,
        ).wait()
        return carry

    lax.fori_loop(0, _NCH, drain, 0)
    plsc.subcore_barrier()
    pltpu.sync_copy(agg_sh.at[pl.ds(s * _RPS, _RPS)],
                    out_hbm.at[c, pl.ds(s * _RPS, _RPS)])


def _msg_body(basis_t_ref, sh_t_ref, xs_ref, w1_ref, b1_ref, w2_ref, b2_ref,
              w3_ref, b3_ref, out_ref):
    f32 = jnp.float32
    basis = jnp.transpose(basis_t_ref[...])
    sh = jnp.transpose(sh_t_ref[...])
    h = jnp.dot(basis, w1_ref[...], preferred_element_type=f32)
    h = h + b1_ref[...]
    h = h * jax.nn.sigmoid(h)
    h = jnp.dot(h, w2_ref[...], preferred_element_type=f32) + b2_ref[...]
    h = h * jax.nn.sigmoid(h)
    tpw = jnp.dot(h, w3_ref[...], preferred_element_type=f32) + b3_ref[...]
    xsh = xs_ref[...] * sh * 0.25
    col_u = lax.broadcasted_iota(jnp.int32, (_MUL, _WN), 1) // _MUL
    row_u = lax.broadcasted_iota(jnp.int32, (_MUL, _WN), 0)
    rmat = (col_u == row_u).astype(f32)
    xsrep = jnp.dot(xsh, rmat, preferred_element_type=f32)
    prod = tpw * xsrep
    col_w = lax.broadcasted_iota(jnp.int32, (_WN, _MUL), 0) % _MUL
    row_w = lax.broadcasted_iota(jnp.int32, (_WN, _MUL), 1)
    smat = (col_w == row_w).astype(f32)
    out_ref[...] = jnp.dot(prod, smat, preferred_element_type=f32)


_tc_messages = pl.pallas_call(
    _msg_body,
    grid=(_EPAD // _BLK,),
    compiler_params=pltpu.CompilerParams(
        allow_input_fusion=[False, False, True, False, False, False, False,
                            False, False]),
    in_specs=[
        pl.BlockSpec((_NB, _BLK), lambda i: (0, i)),
        pl.BlockSpec((1, _BLK), lambda i: (0, i)),
        pl.BlockSpec((_BLK, _MUL), lambda i: (i, 0)),
        pl.BlockSpec((_NB, _HID), lambda i: (0, 0)),
        pl.BlockSpec((1, _HID), lambda i: (0, 0)),
        pl.BlockSpec((_HID, _HID), lambda i: (0, 0)),
        pl.BlockSpec((1, _HID), lambda i: (0, 0)),
        pl.BlockSpec((_HID, _WN), lambda i: (0, 0)),
        pl.BlockSpec((1, _WN), lambda i: (0, 0)),
    ],
    out_specs=pl.BlockSpec((_BLK, _MUL), lambda i: (i, 0)),
    out_shape=jax.ShapeDtypeStruct((_EPAD, _MUL), jnp.float32),
)


def _final_body(p_ref, nf_ref, wsi_ref, bnw_ref, bnb_ref, out_ref):
    agg = (p_ref[0] + p_ref[1])[:_N_NODES]
    agg = agg * jax.nn.sigmoid(agg)
    o = jnp.dot(agg, wsi_ref[...], preferred_element_type=jnp.float32) * 0.25
    mean = jnp.mean(o, axis=0, keepdims=True)
    cen = o - mean
    var = jnp.mean(cen * cen, axis=0, keepdims=True)
    o = cen * lax.rsqrt(var + 1e-5)
    out_ref[...] = o * bnw_ref[...] + bnb_ref[...] + nf_ref[...]


_tc_final = pl.pallas_call(
    _final_body,
    in_specs=[
        pl.BlockSpec((2, _NPAD, _MUL), lambda: (0, 0, 0)),
        pl.BlockSpec((_N_NODES, _MUL), lambda: (0, 0)),
        pl.BlockSpec((_MUL, _MUL), lambda: (0, 0)),
        pl.BlockSpec((1, _MUL), lambda: (0, 0)),
        pl.BlockSpec((1, _MUL), lambda: (0, 0)),
    ],
    out_specs=pl.BlockSpec((_N_NODES, _MUL), lambda: (0, 0)),
    out_shape=jax.ShapeDtypeStruct((_N_NODES, _MUL), jnp.float32),
)


def kernel(node_features, edge_index, edge_sh, edge_basis, W1, b1, W2, b2,
           W3, b3, W_si, bn_weight, bn_bias):
    pad = _EPAD - _E
    src = jnp.pad(edge_index[0], (0, pad)).reshape(_NW, _EPW)
    dst = jnp.pad(edge_index[1], (0, pad),
                  constant_values=_NPAD - 1).reshape(_NW, _NCH, _CHUNK)
    sh_t = jnp.pad(edge_sh.T, ((0, 0), (0, pad)))
    basis_t = jnp.pad(edge_basis.T, ((0, 0), (0, pad)))

    x_src = _build_sc_gather()(node_features, src).reshape(_EPAD, _MUL)
    msgs = _tc_messages(basis_t, sh_t, x_src,
                        W1, b1.reshape(1, _HID), W2, b2.reshape(1, _HID),
                        W3, b3.reshape(1, _WN))
    partials = _build_sc_scatter()(msgs.reshape(_NW, _EPW, _MUL), dst,
                                   jnp.zeros((_NPAD, _MUL), jnp.float32))
    return _tc_final(partials, node_features, W_si,
                     bn_weight.reshape(1, _MUL), bn_bias.reshape(1, _MUL))


# speedup vs baseline: 3.4080x; 1.0016x over previous
import functools

import jax
import jax.numpy as jnp
from jax import lax
from jax.experimental import pallas as pl
from jax.experimental.pallas import tpu as pltpu
from jax.experimental.pallas import tpu_sc as plsc

_MUL = 16
_NB = 8
_HID = 64
_WN = 256
_N_NODES = 10000
_E = 160000

_NW = 32
_CHUNK = 128
_NCH = 40
_EPW = _CHUNK * _NCH
_EPAD = _NW * _EPW
_BLK = 2048
_RPS = 640
_NPAD = 16 * _RPS

@functools.cache
def _build_sc_gather():
    mesh = plsc.VectorSubcoreMesh(core_axis_name="c", subcore_axis_name="s")
    return functools.partial(
        pl.kernel,
        out_type=jax.ShapeDtypeStruct((_NW, _EPW, _MUL), jnp.float32),
        mesh=mesh,
        compiler_params=pltpu.CompilerParams(use_tc_tiling_on_sc=False),
        scratch_types=[
            pltpu.VMEM((_EPW,), jnp.int32),
            pltpu.VMEM((_EPW, _MUL), jnp.float32),
            pltpu.SemaphoreType.DMA,
        ],
    )(_sc_gather_body)


def _sc_gather_body(node_hbm, src_hbm, x_hbm, idx_v, xbuf, sem):
    c = lax.axis_index("c")
    s = lax.axis_index("s")
    wid = s * 2 + c
    pltpu.sync_copy(src_hbm.at[wid], idx_v)
    pltpu.async_copy(node_hbm.at[idx_v], xbuf, sem).wait()
    pltpu.sync_copy(xbuf, x_hbm.at[wid])


@functools.cache
def _build_sc_scatter():
    mesh = plsc.VectorSubcoreMesh(core_axis_name="c", subcore_axis_name="s")
    return functools.partial(
        pl.kernel,
        out_type=jax.ShapeDtypeStruct((2, _NPAD, _MUL), jnp.float32),
        mesh=mesh,
        compiler_params=pltpu.CompilerParams(use_tc_tiling_on_sc=False),
        scratch_types=[
            pltpu.VMEM((_NCH, _CHUNK), jnp.int32),
            pltpu.VMEM((_EPW, _MUL), jnp.float32),
            pltpu.VMEM_SHARED((_NPAD, _MUL), jnp.float32),
            pltpu.SemaphoreType.DMA,
        ],
    )(_sc_scatter_body)


def _sc_scatter_body(msg_hbm, dst_hbm, zeros_hbm, out_hbm, idx_v, mbuf, agg_sh, sem):
    c = lax.axis_index("c")
    s = lax.axis_index("s")
    wid = s * 2 + c
    pltpu.sync_copy(zeros_hbm.at[pl.ds(s * _RPS, _RPS)],
                    agg_sh.at[pl.ds(s * _RPS, _RPS)])
    pltpu.sync_copy(dst_hbm.at[wid], idx_v)
    pltpu.sync_copy(msg_hbm.at[wid], mbuf)
    plsc.subcore_barrier()

    def fire(j, carry):
        pltpu.async_copy(
            mbuf.at[pl.ds(j * _CHUNK, _CHUNK)],
            agg_sh.at[idx_v.at[j]],
            sem,
            add=True,
        )
        return carry

    lax.fori_loop(0, _NCH, fire, 0)
    pltpu.make_async_copy(msg_hbm.at[wid], mbuf, sem).wait()
    plsc.subcore_barrier()
    pltpu.sync_copy(agg_sh.at[pl.ds(s * _RPS, _RPS)],
                    out_hbm.at[c, pl.ds(s * _RPS, _RPS)])


def _msg_body(basis_t_ref, sh_t_ref, xs_ref, w1_ref, b1_ref, w2_ref, b2_ref,
              w3_ref, b3_ref, out_ref):
    f32 = jnp.float32
    basis = jnp.transpose(basis_t_ref[...])
    sh = jnp.transpose(sh_t_ref[...])
    h = jnp.dot(basis, w1_ref[...], preferred_element_type=f32)
    h = h + b1_ref[...]
    h = h * jax.nn.sigmoid(h)
    h = jnp.dot(h, w2_ref[...], preferred_element_type=f32) + b2_ref[...]
    h = h * jax.nn.sigmoid(h)
    tpw = jnp.dot(h, w3_ref[...], preferred_element_type=f32) + b3_ref[...]
    xsh = xs_ref[...] * sh * 0.25
    col_u = lax.broadcasted_iota(jnp.int32, (_MUL, _WN), 1) // _MUL
    row_u = lax.broadcasted_iota(jnp.int32, (_MUL, _WN), 0)
    rmat = (col_u == row_u).astype(f32)
    xsrep = jnp.dot(xsh, rmat, preferred_element_type=f32)
    prod = tpw * xsrep
    col_w = lax.broadcasted_iota(jnp.int32, (_WN, _MUL), 0) % _MUL
    row_w = lax.broadcasted_iota(jnp.int32, (_WN, _MUL), 1)
    smat = (col_w == row_w).astype(f32)
    out_ref[...] = jnp.dot(prod, smat, preferred_element_type=f32)


_tc_messages = pl.pallas_call(
    _msg_body,
    grid=(_EPAD // _BLK,),
    compiler_params=pltpu.CompilerParams(
        allow_input_fusion=[False, False, True, False, False, False, False,
                            False, False]),
    in_specs=[
        pl.BlockSpec((_NB, _BLK), lambda i: (0, i)),
        pl.BlockSpec((1, _BLK), lambda i: (0, i)),
        pl.BlockSpec((_BLK, _MUL), lambda i: (i, 0)),
        pl.BlockSpec((_NB, _HID), lambda i: (0, 0)),
        pl.BlockSpec((1, _HID), lambda i: (0, 0)),
        pl.BlockSpec((_HID, _HID), lambda i: (0, 0)),
        pl.BlockSpec((1, _HID), lambda i: (0, 0)),
        pl.BlockSpec((_HID, _WN), lambda i: (0, 0)),
        pl.BlockSpec((1, _WN), lambda i: (0, 0)),
    ],
    out_specs=pl.BlockSpec((_BLK, _MUL), lambda i: (i, 0)),
    out_shape=jax.ShapeDtypeStruct((_EPAD, _MUL), jnp.float32),
)


def _final_body(p_ref, nf_ref, wsi_ref, bnw_ref, bnb_ref, out_ref):
    agg = (p_ref[0] + p_ref[1])[:_N_NODES]
    agg = agg * jax.nn.sigmoid(agg)
    o = jnp.dot(agg, wsi_ref[...], preferred_element_type=jnp.float32) * 0.25
    mean = jnp.mean(o, axis=0, keepdims=True)
    cen = o - mean
    var = jnp.mean(cen * cen, axis=0, keepdims=True)
    o = cen * lax.rsqrt(var + 1e-5)
    out_ref[...] = o * bnw_ref[...] + bnb_ref[...] + nf_ref[...]


_tc_final = pl.pallas_call(
    _final_body,
    in_specs=[
        pl.BlockSpec((2, _NPAD, _MUL), lambda: (0, 0, 0)),
        pl.BlockSpec((_N_NODES, _MUL), lambda: (0, 0)),
        pl.BlockSpec((_MUL, _MUL), lambda: (0, 0)),
        pl.BlockSpec((1, _MUL), lambda: (0, 0)),
        pl.BlockSpec((1, _MUL), lambda: (0, 0)),
    ],
    out_specs=pl.BlockSpec((_N_NODES, _MUL), lambda: (0, 0)),
    out_shape=jax.ShapeDtypeStruct((_N_NODES, _MUL), jnp.float32),
)


def kernel(node_features, edge_index, edge_sh, edge_basis, W1, b1, W2, b2,
           W3, b3, W_si, bn_weight, bn_bias):
    pad = _EPAD - _E
    src = jnp.pad(edge_index[0], (0, pad)).reshape(_NW, _EPW)
    dst = jnp.pad(edge_index[1], (0, pad),
                  constant_values=_NPAD - 1).reshape(_NW, _NCH, _CHUNK)
    sh_t = jnp.pad(edge_sh.T, ((0, 0), (0, pad)))
    basis_t = jnp.pad(edge_basis.T, ((0, 0), (0, pad)))

    x_src = _build_sc_gather()(node_features, src).reshape(_EPAD, _MUL)
    msgs = _tc_messages(basis_t, sh_t, x_src,
                        W1, b1.reshape(1, _HID), W2, b2.reshape(1, _HID),
                        W3, b3.reshape(1, _WN))
    partials = _build_sc_scatter()(msgs.reshape(_NW, _EPW, _MUL), dst,
                                   jnp.zeros((_NPAD, _MUL), jnp.float32))
    return _tc_final(partials, node_features, W_si,
                     bn_weight.reshape(1, _MUL), bn_bias.reshape(1, _MUL))


# speedup vs baseline: 3.5601x; 1.0446x over previous
import functools

import jax
import jax.numpy as jnp
from jax import lax
from jax.experimental import pallas as pl
from jax.experimental.pallas import tpu as pltpu
from jax.experimental.pallas import tpu_sc as plsc

_MUL = 16
_NB = 8
_HID = 64
_WN = 256
_N_NODES = 10000
_E = 160000

_NW = 32
_CHUNK = 128
_NCH = 40
_EPW = _CHUNK * _NCH
_EPAD = _NW * _EPW
_BLK = 4096
_RPS = 640
_NPAD = 16 * _RPS

@functools.cache
def _build_sc_gather():
    mesh = plsc.VectorSubcoreMesh(core_axis_name="c", subcore_axis_name="s")
    return functools.partial(
        pl.kernel,
        out_type=jax.ShapeDtypeStruct((_NW, _EPW, _MUL), jnp.float32),
        mesh=mesh,
        compiler_params=pltpu.CompilerParams(use_tc_tiling_on_sc=False),
        scratch_types=[
            pltpu.VMEM((_EPW,), jnp.int32),
            pltpu.VMEM((_EPW, _MUL), jnp.float32),
            pltpu.SemaphoreType.DMA,
        ],
    )(_sc_gather_body)


def _sc_gather_body(node_hbm, src_hbm, x_hbm, idx_v, xbuf, sem):
    c = lax.axis_index("c")
    s = lax.axis_index("s")
    wid = s * 2 + c
    pltpu.sync_copy(src_hbm.at[wid], idx_v)
    pltpu.async_copy(node_hbm.at[idx_v], xbuf, sem).wait()
    pltpu.sync_copy(xbuf, x_hbm.at[wid])


@functools.cache
def _build_sc_scatter():
    mesh = plsc.VectorSubcoreMesh(core_axis_name="c", subcore_axis_name="s")
    return functools.partial(
        pl.kernel,
        out_type=jax.ShapeDtypeStruct((2, _NPAD, _MUL), jnp.float32),
        mesh=mesh,
        compiler_params=pltpu.CompilerParams(use_tc_tiling_on_sc=False),
        scratch_types=[
            pltpu.VMEM((_NCH, _CHUNK), jnp.int32),
            pltpu.VMEM((_EPW, _MUL), jnp.float32),
            pltpu.VMEM_SHARED((_NPAD, _MUL), jnp.float32),
            pltpu.SemaphoreType.DMA,
        ],
    )(_sc_scatter_body)


def _sc_scatter_body(msg_hbm, dst_hbm, zeros_hbm, out_hbm, idx_v, mbuf, agg_sh, sem):
    c = lax.axis_index("c")
    s = lax.axis_index("s")
    wid = s * 2 + c
    pltpu.sync_copy(zeros_hbm.at[pl.ds(s * _RPS, _RPS)],
                    agg_sh.at[pl.ds(s * _RPS, _RPS)])
    pltpu.sync_copy(dst_hbm.at[wid], idx_v)
    pltpu.sync_copy(msg_hbm.at[wid], mbuf)
    plsc.subcore_barrier()

    def fire(j, carry):
        pltpu.async_copy(
            mbuf.at[pl.ds(j * _CHUNK, _CHUNK)],
            agg_sh.at[idx_v.at[j]],
            sem,
            add=True,
        )
        return carry

    lax.fori_loop(0, _NCH, fire, 0)
    pltpu.make_async_copy(msg_hbm.at[wid], mbuf, sem).wait()
    plsc.subcore_barrier()
    pltpu.sync_copy(agg_sh.at[pl.ds(s * _RPS, _RPS)],
                    out_hbm.at[c, pl.ds(s * _RPS, _RPS)])


def _msg_body(basis_t_ref, sh_t_ref, xs_ref, w1_ref, b1_ref, w2_ref, b2_ref,
              w3_ref, b3_ref, out_ref):
    f32 = jnp.float32
    basis = jnp.transpose(basis_t_ref[...])
    sh = jnp.transpose(sh_t_ref[...])
    h = jnp.dot(basis, w1_ref[...], preferred_element_type=f32)
    h = h + b1_ref[...]
    h = h * jax.nn.sigmoid(h)
    h = jnp.dot(h, w2_ref[...], preferred_element_type=f32) + b2_ref[...]
    h = h * jax.nn.sigmoid(h)
    tpw = jnp.dot(h, w3_ref[...], preferred_element_type=f32) + b3_ref[...]
    xsh = xs_ref[...] * sh * 0.25
    col_u = lax.broadcasted_iota(jnp.int32, (_MUL, _WN), 1) // _MUL
    row_u = lax.broadcasted_iota(jnp.int32, (_MUL, _WN), 0)
    rmat = (col_u == row_u).astype(f32)
    xsrep = jnp.dot(xsh, rmat, preferred_element_type=f32)
    prod = tpw * xsrep
    col_w = lax.broadcasted_iota(jnp.int32, (_WN, _MUL), 0) % _MUL
    row_w = lax.broadcasted_iota(jnp.int32, (_WN, _MUL), 1)
    smat = (col_w == row_w).astype(f32)
    out_ref[...] = jnp.dot(prod, smat, preferred_element_type=f32)


_tc_messages = pl.pallas_call(
    _msg_body,
    grid=(_EPAD // _BLK,),
    compiler_params=pltpu.CompilerParams(
        allow_input_fusion=[False, False, True, False, False, False, False,
                            False, False]),
    in_specs=[
        pl.BlockSpec((_NB, _BLK), lambda i: (0, i)),
        pl.BlockSpec((1, _BLK), lambda i: (0, i)),
        pl.BlockSpec((_BLK, _MUL), lambda i: (i, 0)),
        pl.BlockSpec((_NB, _HID), lambda i: (0, 0)),
        pl.BlockSpec((1, _HID), lambda i: (0, 0)),
        pl.BlockSpec((_HID, _HID), lambda i: (0, 0)),
        pl.BlockSpec((1, _HID), lambda i: (0, 0)),
        pl.BlockSpec((_HID, _WN), lambda i: (0, 0)),
        pl.BlockSpec((1, _WN), lambda i: (0, 0)),
    ],
    out_specs=pl.BlockSpec((_BLK, _MUL), lambda i: (i, 0)),
    out_shape=jax.ShapeDtypeStruct((_EPAD, _MUL), jnp.float32),
)


def _final_body(p_ref, nf_ref, wsi_ref, bnw_ref, bnb_ref, out_ref):
    agg = (p_ref[0] + p_ref[1])[:_N_NODES]
    agg = agg * jax.nn.sigmoid(agg)
    o = jnp.dot(agg, wsi_ref[...], preferred_element_type=jnp.float32) * 0.25
    mean = jnp.mean(o, axis=0, keepdims=True)
    cen = o - mean
    var = jnp.mean(cen * cen, axis=0, keepdims=True)
    o = cen * lax.rsqrt(var + 1e-5)
    out_ref[...] = o * bnw_ref[...] + bnb_ref[...] + nf_ref[...]


_tc_final = pl.pallas_call(
    _final_body,
    in_specs=[
        pl.BlockSpec((2, _NPAD, _MUL), lambda: (0, 0, 0)),
        pl.BlockSpec((_N_NODES, _MUL), lambda: (0, 0)),
        pl.BlockSpec((_MUL, _MUL), lambda: (0, 0)),
        pl.BlockSpec((1, _MUL), lambda: (0, 0)),
        pl.BlockSpec((1, _MUL), lambda: (0, 0)),
    ],
    out_specs=pl.BlockSpec((_N_NODES, _MUL), lambda: (0, 0)),
    out_shape=jax.ShapeDtypeStruct((_N_NODES, _MUL), jnp.float32),
)


def kernel(node_features, edge_index, edge_sh, edge_basis, W1, b1, W2, b2,
           W3, b3, W_si, bn_weight, bn_bias):
    pad = _EPAD - _E
    src = jnp.pad(edge_index[0], (0, pad)).reshape(_NW, _EPW)
    dst = jnp.pad(edge_index[1], (0, pad),
                  constant_values=_NPAD - 1).reshape(_NW, _NCH, _CHUNK)
    sh_t = jnp.pad(edge_sh.T, ((0, 0), (0, pad)))
    basis_t = jnp.pad(edge_basis.T, ((0, 0), (0, pad)))

    x_src = _build_sc_gather()(node_features, src).reshape(_EPAD, _MUL)
    msgs = _tc_messages(basis_t, sh_t, x_src,
                        W1, b1.reshape(1, _HID), W2, b2.reshape(1, _HID),
                        W3, b3.reshape(1, _WN))
    partials = _build_sc_scatter()(msgs.reshape(_NW, _EPW, _MUL), dst,
                                   jnp.zeros((_NPAD, _MUL), jnp.float32))
    return _tc_final(partials, node_features, W_si,
                     bn_weight.reshape(1, _MUL), bn_bias.reshape(1, _MUL))


# speedup vs baseline: 3.6512x; 1.0256x over previous
import functools

import jax
import jax.numpy as jnp
from jax import lax
from jax.experimental import pallas as pl
from jax.experimental.pallas import tpu as pltpu
from jax.experimental.pallas import tpu_sc as plsc

_MUL = 16
_NB = 8
_HID = 64
_WN = 256
_N_NODES = 10000
_E = 160000

_NW = 32
_CHUNK = 128
_NCH = 40
_EPW = _CHUNK * _NCH
_EPAD = _NW * _EPW
_BLK = 8192
_RPS = 640
_NPAD = 16 * _RPS

@functools.cache
def _build_sc_gather():
    mesh = plsc.VectorSubcoreMesh(core_axis_name="c", subcore_axis_name="s")
    return functools.partial(
        pl.kernel,
        out_type=jax.ShapeDtypeStruct((_NW, _EPW, _MUL), jnp.float32),
        mesh=mesh,
        compiler_params=pltpu.CompilerParams(use_tc_tiling_on_sc=False),
        scratch_types=[
            pltpu.VMEM((_EPW,), jnp.int32),
            pltpu.VMEM((_EPW, _MUL), jnp.float32),
            pltpu.SemaphoreType.DMA,
        ],
    )(_sc_gather_body)


def _sc_gather_body(node_hbm, src_hbm, x_hbm, idx_v, xbuf, sem):
    c = lax.axis_index("c")
    s = lax.axis_index("s")
    wid = s * 2 + c
    pltpu.sync_copy(src_hbm.at[wid], idx_v)
    pltpu.async_copy(node_hbm.at[idx_v], xbuf, sem).wait()
    pltpu.sync_copy(xbuf, x_hbm.at[wid])


@functools.cache
def _build_sc_scatter():
    mesh = plsc.VectorSubcoreMesh(core_axis_name="c", subcore_axis_name="s")
    return functools.partial(
        pl.kernel,
        out_type=jax.ShapeDtypeStruct((2, _NPAD, _MUL), jnp.float32),
        mesh=mesh,
        compiler_params=pltpu.CompilerParams(use_tc_tiling_on_sc=False),
        scratch_types=[
            pltpu.VMEM((_NCH, _CHUNK), jnp.int32),
            pltpu.VMEM((_EPW, _MUL), jnp.float32),
            pltpu.VMEM_SHARED((_NPAD, _MUL), jnp.float32),
            pltpu.SemaphoreType.DMA,
        ],
    )(_sc_scatter_body)


def _sc_scatter_body(msg_hbm, dst_hbm, zeros_hbm, out_hbm, idx_v, mbuf, agg_sh, sem):
    c = lax.axis_index("c")
    s = lax.axis_index("s")
    wid = s * 2 + c
    pltpu.sync_copy(zeros_hbm.at[pl.ds(s * _RPS, _RPS)],
                    agg_sh.at[pl.ds(s * _RPS, _RPS)])
    pltpu.sync_copy(dst_hbm.at[wid], idx_v)
    pltpu.sync_copy(msg_hbm.at[wid], mbuf)
    plsc.subcore_barrier()

    def fire(j, carry):
        pltpu.async_copy(
            mbuf.at[pl.ds(j * _CHUNK, _CHUNK)],
            agg_sh.at[idx_v.at[j]],
            sem,
            add=True,
        )
        return carry

    lax.fori_loop(0, _NCH, fire, 0)
    pltpu.make_async_copy(msg_hbm.at[wid], mbuf, sem).wait()
    plsc.subcore_barrier()
    pltpu.sync_copy(agg_sh.at[pl.ds(s * _RPS, _RPS)],
                    out_hbm.at[c, pl.ds(s * _RPS, _RPS)])


def _msg_body(basis_t_ref, sh_t_ref, xs_ref, w1_ref, b1_ref, w2_ref, b2_ref,
              w3_ref, b3_ref, out_ref):
    f32 = jnp.float32
    basis = jnp.transpose(basis_t_ref[...])
    sh = jnp.transpose(sh_t_ref[...])
    h = jnp.dot(basis, w1_ref[...], preferred_element_type=f32)
    h = h + b1_ref[...]
    h = h * jax.nn.sigmoid(h)
    h = jnp.dot(h, w2_ref[...], preferred_element_type=f32) + b2_ref[...]
    h = h * jax.nn.sigmoid(h)
    tpw = jnp.dot(h, w3_ref[...], preferred_element_type=f32) + b3_ref[...]
    xsh = xs_ref[...] * sh * 0.25
    col_u = lax.broadcasted_iota(jnp.int32, (_MUL, _WN), 1) // _MUL
    row_u = lax.broadcasted_iota(jnp.int32, (_MUL, _WN), 0)
    rmat = (col_u == row_u).astype(f32)
    xsrep = jnp.dot(xsh, rmat, preferred_element_type=f32)
    prod = tpw * xsrep
    col_w = lax.broadcasted_iota(jnp.int32, (_WN, _MUL), 0) % _MUL
    row_w = lax.broadcasted_iota(jnp.int32, (_WN, _MUL), 1)
    smat = (col_w == row_w).astype(f32)
    out_ref[...] = jnp.dot(prod, smat, preferred_element_type=f32)


_tc_messages = pl.pallas_call(
    _msg_body,
    grid=(_EPAD // _BLK,),
    compiler_params=pltpu.CompilerParams(
        allow_input_fusion=[False, False, True, False, False, False, False,
                            False, False]),
    in_specs=[
        pl.BlockSpec((_NB, _BLK), lambda i: (0, i)),
        pl.BlockSpec((1, _BLK), lambda i: (0, i)),
        pl.BlockSpec((_BLK, _MUL), lambda i: (i, 0)),
        pl.BlockSpec((_NB, _HID), lambda i: (0, 0)),
        pl.BlockSpec((1, _HID), lambda i: (0, 0)),
        pl.BlockSpec((_HID, _HID), lambda i: (0, 0)),
        pl.BlockSpec((1, _HID), lambda i: (0, 0)),
        pl.BlockSpec((_HID, _WN), lambda i: (0, 0)),
        pl.BlockSpec((1, _WN), lambda i: (0, 0)),
    ],
    out_specs=pl.BlockSpec((_BLK, _MUL), lambda i: (i, 0)),
    out_shape=jax.ShapeDtypeStruct((_EPAD, _MUL), jnp.float32),
)


def _final_body(p_ref, nf_ref, wsi_ref, bnw_ref, bnb_ref, out_ref):
    agg = (p_ref[0] + p_ref[1])[:_N_NODES]
    agg = agg * jax.nn.sigmoid(agg)
    o = jnp.dot(agg, wsi_ref[...], preferred_element_type=jnp.float32) * 0.25
    mean = jnp.mean(o, axis=0, keepdims=True)
    cen = o - mean
    var = jnp.mean(cen * cen, axis=0, keepdims=True)
    o = cen * lax.rsqrt(var + 1e-5)
    out_ref[...] = o * bnw_ref[...] + bnb_ref[...] + nf_ref[...]


_tc_final = pl.pallas_call(
    _final_body,
    in_specs=[
        pl.BlockSpec((2, _NPAD, _MUL), lambda: (0, 0, 0)),
        pl.BlockSpec((_N_NODES, _MUL), lambda: (0, 0)),
        pl.BlockSpec((_MUL, _MUL), lambda: (0, 0)),
        pl.BlockSpec((1, _MUL), lambda: (0, 0)),
        pl.BlockSpec((1, _MUL), lambda: (0, 0)),
    ],
    out_specs=pl.BlockSpec((_N_NODES, _MUL), lambda: (0, 0)),
    out_shape=jax.ShapeDtypeStruct((_N_NODES, _MUL), jnp.float32),
)


def kernel(node_features, edge_index, edge_sh, edge_basis, W1, b1, W2, b2,
           W3, b3, W_si, bn_weight, bn_bias):
    pad = _EPAD - _E
    src = jnp.pad(edge_index[0], (0, pad)).reshape(_NW, _EPW)
    dst = jnp.pad(edge_index[1], (0, pad),
                  constant_values=_NPAD - 1).reshape(_NW, _NCH, _CHUNK)
    sh_t = jnp.pad(edge_sh.T, ((0, 0), (0, pad)))
    basis_t = jnp.pad(edge_basis.T, ((0, 0), (0, pad)))

    x_src = _build_sc_gather()(node_features, src).reshape(_EPAD, _MUL)
    msgs = _tc_messages(basis_t, sh_t, x_src,
                        W1, b1.reshape(1, _HID), W2, b2.reshape(1, _HID),
                        W3, b3.reshape(1, _WN))
    partials = _build_sc_scatter()(msgs.reshape(_NW, _EPW, _MUL), dst,
                                   jnp.zeros((_NPAD, _MUL), jnp.float32))
    return _tc_final(partials, node_features, W_si,
                     bn_weight.reshape(1, _MUL), bn_bias.reshape(1, _MUL))


# speedup vs baseline: 3.6857x; 1.0094x over previous
import functools

import jax
import jax.numpy as jnp
from jax import lax
from jax.experimental import pallas as pl
from jax.experimental.pallas import tpu as pltpu
from jax.experimental.pallas import tpu_sc as plsc

_MUL = 16
_NB = 8
_HID = 64
_WN = 256
_N_NODES = 10000
_E = 160000

_NW = 32
_CHUNK = 128
_NCH = 40
_EPW = _CHUNK * _NCH
_EPAD = _NW * _EPW
_BLK = 16384
_RPS = 640
_NPAD = 16 * _RPS

@functools.cache
def _build_sc_gather():
    mesh = plsc.VectorSubcoreMesh(core_axis_name="c", subcore_axis_name="s")
    return functools.partial(
        pl.kernel,
        out_type=jax.ShapeDtypeStruct((_NW, _EPW, _MUL), jnp.float32),
        mesh=mesh,
        compiler_params=pltpu.CompilerParams(use_tc_tiling_on_sc=False),
        scratch_types=[
            pltpu.VMEM((_EPW,), jnp.int32),
            pltpu.VMEM((_EPW, _MUL), jnp.float32),
            pltpu.SemaphoreType.DMA,
        ],
    )(_sc_gather_body)


def _sc_gather_body(node_hbm, src_hbm, x_hbm, idx_v, xbuf, sem):
    c = lax.axis_index("c")
    s = lax.axis_index("s")
    wid = s * 2 + c
    pltpu.sync_copy(src_hbm.at[wid], idx_v)
    pltpu.async_copy(node_hbm.at[idx_v], xbuf, sem).wait()
    pltpu.sync_copy(xbuf, x_hbm.at[wid])


@functools.cache
def _build_sc_scatter():
    mesh = plsc.VectorSubcoreMesh(core_axis_name="c", subcore_axis_name="s")
    return functools.partial(
        pl.kernel,
        out_type=jax.ShapeDtypeStruct((2, _NPAD, _MUL), jnp.float32),
        mesh=mesh,
        compiler_params=pltpu.CompilerParams(use_tc_tiling_on_sc=False),
        scratch_types=[
            pltpu.VMEM((_NCH, _CHUNK), jnp.int32),
            pltpu.VMEM((_EPW, _MUL), jnp.float32),
            pltpu.VMEM_SHARED((_NPAD, _MUL), jnp.float32),
            pltpu.SemaphoreType.DMA,
        ],
    )(_sc_scatter_body)


def _sc_scatter_body(msg_hbm, dst_hbm, zeros_hbm, out_hbm, idx_v, mbuf, agg_sh, sem):
    c = lax.axis_index("c")
    s = lax.axis_index("s")
    wid = s * 2 + c
    pltpu.sync_copy(zeros_hbm.at[pl.ds(s * _RPS, _RPS)],
                    agg_sh.at[pl.ds(s * _RPS, _RPS)])
    pltpu.sync_copy(dst_hbm.at[wid], idx_v)
    pltpu.sync_copy(msg_hbm.at[wid], mbuf)
    plsc.subcore_barrier()

    def fire(j, carry):
        pltpu.async_copy(
            mbuf.at[pl.ds(j * _CHUNK, _CHUNK)],
            agg_sh.at[idx_v.at[j]],
            sem,
            add=True,
        )
        return carry

    lax.fori_loop(0, _NCH, fire, 0)
    pltpu.make_async_copy(msg_hbm.at[wid], mbuf, sem).wait()
    plsc.subcore_barrier()
    pltpu.sync_copy(agg_sh.at[pl.ds(s * _RPS, _RPS)],
                    out_hbm.at[c, pl.ds(s * _RPS, _RPS)])


def _msg_body(basis_t_ref, sh_t_ref, xs_ref, w1_ref, b1_ref, w2_ref, b2_ref,
              w3_ref, b3_ref, out_ref):
    f32 = jnp.float32
    basis = jnp.transpose(basis_t_ref[...])
    sh = jnp.transpose(sh_t_ref[...])
    h = jnp.dot(basis, w1_ref[...], preferred_element_type=f32)
    h = h + b1_ref[...]
    h = h * jax.nn.sigmoid(h)
    h = jnp.dot(h, w2_ref[...], preferred_element_type=f32) + b2_ref[...]
    h = h * jax.nn.sigmoid(h)
    tpw = jnp.dot(h, w3_ref[...], preferred_element_type=f32) + b3_ref[...]
    xsh = xs_ref[...] * sh * 0.25
    col_u = lax.broadcasted_iota(jnp.int32, (_MUL, _WN), 1) // _MUL
    row_u = lax.broadcasted_iota(jnp.int32, (_MUL, _WN), 0)
    rmat = (col_u == row_u).astype(f32)
    xsrep = jnp.dot(xsh, rmat, preferred_element_type=f32)
    prod = tpw * xsrep
    col_w = lax.broadcasted_iota(jnp.int32, (_WN, _MUL), 0) % _MUL
    row_w = lax.broadcasted_iota(jnp.int32, (_WN, _MUL), 1)
    smat = (col_w == row_w).astype(f32)
    out_ref[...] = jnp.dot(prod, smat, preferred_element_type=f32)


_tc_messages = pl.pallas_call(
    _msg_body,
    grid=(_EPAD // _BLK,),
    compiler_params=pltpu.CompilerParams(
        allow_input_fusion=[False, False, True, False, False, False, False,
                            False, False]),
    in_specs=[
        pl.BlockSpec((_NB, _BLK), lambda i: (0, i)),
        pl.BlockSpec((1, _BLK), lambda i: (0, i)),
        pl.BlockSpec((_BLK, _MUL), lambda i: (i, 0)),
        pl.BlockSpec((_NB, _HID), lambda i: (0, 0)),
        pl.BlockSpec((1, _HID), lambda i: (0, 0)),
        pl.BlockSpec((_HID, _HID), lambda i: (0, 0)),
        pl.BlockSpec((1, _HID), lambda i: (0, 0)),
        pl.BlockSpec((_HID, _WN), lambda i: (0, 0)),
        pl.BlockSpec((1, _WN), lambda i: (0, 0)),
    ],
    out_specs=pl.BlockSpec((_BLK, _MUL), lambda i: (i, 0)),
    out_shape=jax.ShapeDtypeStruct((_EPAD, _MUL), jnp.float32),
)


def _final_body(p_ref, nf_ref, wsi_ref, bnw_ref, bnb_ref, out_ref):
    agg = (p_ref[0] + p_ref[1])[:_N_NODES]
    agg = agg * jax.nn.sigmoid(agg)
    o = jnp.dot(agg, wsi_ref[...], preferred_element_type=jnp.float32) * 0.25
    mean = jnp.mean(o, axis=0, keepdims=True)
    cen = o - mean
    var = jnp.mean(cen * cen, axis=0, keepdims=True)
    o = cen * lax.rsqrt(var + 1e-5)
    out_ref[...] = o * bnw_ref[...] + bnb_ref[...] + nf_ref[...]


_tc_final = pl.pallas_call(
    _final_body,
    in_specs=[
        pl.BlockSpec((2, _NPAD, _MUL), lambda: (0, 0, 0)),
        pl.BlockSpec((_N_NODES, _MUL), lambda: (0, 0)),
        pl.BlockSpec((_MUL, _MUL), lambda: (0, 0)),
        pl.BlockSpec((1, _MUL), lambda: (0, 0)),
        pl.BlockSpec((1, _MUL), lambda: (0, 0)),
    ],
    out_specs=pl.BlockSpec((_N_NODES, _MUL), lambda: (0, 0)),
    out_shape=jax.ShapeDtypeStruct((_N_NODES, _MUL), jnp.float32),
)


def kernel(node_features, edge_index, edge_sh, edge_basis, W1, b1, W2, b2,
           W3, b3, W_si, bn_weight, bn_bias):
    pad = _EPAD - _E
    src = jnp.pad(edge_index[0], (0, pad)).reshape(_NW, _EPW)
    dst = jnp.pad(edge_index[1], (0, pad),
                  constant_values=_NPAD - 1).reshape(_NW, _NCH, _CHUNK)
    sh_t = jnp.pad(edge_sh.T, ((0, 0), (0, pad)))
    basis_t = jnp.pad(edge_basis.T, ((0, 0), (0, pad)))

    x_src = _build_sc_gather()(node_features, src).reshape(_EPAD, _MUL)
    msgs = _tc_messages(basis_t, sh_t, x_src,
                        W1, b1.reshape(1, _HID), W2, b2.reshape(1, _HID),
                        W3, b3.reshape(1, _WN))
    partials = _build_sc_scatter()(msgs.reshape(_NW, _EPW, _MUL), dst,
                                   jnp.zeros((_NPAD, _MUL), jnp.float32))
    return _tc_final(partials, node_features, W_si,
                     bn_weight.reshape(1, _MUL), bn_bias.reshape(1, _MUL))
